# baseline (speedup 1.0000x reference)
import functools

import jax
import jax.numpy as jnp
from jax import lax
from jax.experimental import pallas as pl
from jax.experimental.pallas import tpu as pltpu
from jax.experimental.pallas import tpu_sc as plsc

NUM_CORES = 2
NUM_SUBCORES = 16
NUM_WORKERS = NUM_CORES * NUM_SUBCORES

CHUNK = 128


def _make_gather(num_ids: int, emb_dim: int):
  assert num_ids % (NUM_WORKERS * CHUNK) == 0
  chunks_per_w = num_ids // (NUM_WORKERS * CHUNK)
  mesh = plsc.VectorSubcoreMesh(core_axis_name="c", subcore_axis_name="s")

  @functools.partial(
      pl.kernel,
      out_type=jax.ShapeDtypeStruct((num_ids, emb_dim), jnp.float32),
      mesh=mesh,
      compiler_params=pltpu.CompilerParams(use_tc_tiling_on_sc=False),
      scratch_types=[
          pltpu.VMEM((chunks_per_w, CHUNK), jnp.int32),
          pltpu.VMEM((CHUNK, emb_dim), jnp.float32),
          pltpu.SemaphoreType.DMA,
      ],
  )
  def gather_kernel(ids_hbm, table_hbm, out_hbm, idx_v, rows_v, gsem):
    wid = lax.axis_index("s") * NUM_CORES + lax.axis_index("c")
    pltpu.sync_copy(ids_hbm.at[wid], idx_v)

    def body(j, carry):
      c = wid * chunks_per_w + j
      pltpu.async_copy(table_hbm.at[idx_v.at[j]], rows_v, gsem).wait()
      pltpu.sync_copy(rows_v, out_hbm.at[pl.ds(c * CHUNK, CHUNK)])
      return carry

    lax.fori_loop(0, chunks_per_w, body, 0)

  return gather_kernel


def kernel(input_ids, attention_mask, weight):
  batch, seq = input_ids.shape
  vocab, emb_dim = weight.shape
  num_ids = batch * seq
  ids3d = input_ids.astype(jnp.int32).reshape(
      NUM_WORKERS, num_ids // (NUM_WORKERS * CHUNK), CHUNK)
  out = _make_gather(num_ids, emb_dim)(ids3d, weight)
  return out.reshape(batch, seq, emb_dim), attention_mask


# speedup vs baseline: 1.0478x; 1.0478x over previous
import functools

import jax
import jax.numpy as jnp
from jax import lax
from jax.experimental import pallas as pl
from jax.experimental.pallas import tpu as pltpu
from jax.experimental.pallas import tpu_sc as plsc

NUM_CORES = 2
NUM_SUBCORES = 16
NUM_WORKERS = NUM_CORES * NUM_SUBCORES

CHUNK = 128
NBUF = 8
INFLIGHT = 4


def _make_gather(num_ids: int, emb_dim: int):
  assert num_ids % (NUM_WORKERS * CHUNK) == 0
  n_chunks = num_ids // (NUM_WORKERS * CHUNK)
  assert n_chunks > NBUF >= INFLIGHT
  mesh = plsc.VectorSubcoreMesh(core_axis_name="c", subcore_axis_name="s")

  @functools.partial(
      pl.kernel,
      out_type=jax.ShapeDtypeStruct((num_ids, emb_dim), jnp.float32),
      mesh=mesh,
      compiler_params=pltpu.CompilerParams(use_tc_tiling_on_sc=False),
      scratch_types=[
          pltpu.VMEM((n_chunks, CHUNK), jnp.int32),
          pltpu.VMEM((NBUF, CHUNK, emb_dim), jnp.float32),
          pltpu.SemaphoreType.DMA((NBUF,)),
          pltpu.SemaphoreType.DMA((NBUF,)),
      ],
  )
  def gather_kernel(ids_hbm, table_hbm, out_hbm, idx_v, rows_v, gsem, ssem):
    wid = lax.axis_index("s") * NUM_CORES + lax.axis_index("c")
    pltpu.sync_copy(ids_hbm.at[wid], idx_v)
    out_base = wid * n_chunks

    def start_gather(c, b):
      pltpu.async_copy(table_hbm.at[idx_v.at[c]], rows_v.at[b], gsem.at[b])

    def wait_gather(b):
      pltpu.make_async_copy(
          out_hbm.at[pl.ds(0, CHUNK)], rows_v.at[b], gsem.at[b]).wait()

    def start_store(c, b):
      pltpu.async_copy(
          rows_v.at[b], out_hbm.at[pl.ds((out_base + c) * CHUNK, CHUNK)],
          ssem.at[b])

    def wait_store(b):
      pltpu.make_async_copy(
          rows_v.at[b], out_hbm.at[pl.ds(0, CHUNK)], ssem.at[b]).wait()

    for c in range(INFLIGHT):
      start_gather(c, c)

    def body(j, carry):
      nxt = j + INFLIGHT

      @pl.when(nxt < n_chunks)
      def _issue():
        b_nxt = nxt % NBUF

        @pl.when(nxt >= NBUF)
        def _reclaim():
          wait_store(b_nxt)

        start_gather(nxt, b_nxt)

      b = j % NBUF
      wait_gather(b)
      start_store(j, b)
      return carry

    lax.fori_loop(0, n_chunks, body, 0)

    for b in range(NBUF):
      wait_store(b)

  return gather_kernel


def kernel(input_ids, attention_mask, weight):
  batch, seq = input_ids.shape
  vocab, emb_dim = weight.shape
  num_ids = batch * seq
  ids3d = input_ids.astype(jnp.int32).reshape(
      NUM_WORKERS, num_ids // (NUM_WORKERS * CHUNK), CHUNK)
  out = _make_gather(num_ids, emb_dim)(ids3d, weight)
  return out.reshape(batch, seq, emb_dim), attention_mask
